# Initial kernel scaffold; baseline (speedup 1.0000x reference)
import jax
import jax.numpy as jnp
from jax.experimental import pallas as pl


def kernel(point_cloud):
    raise NotImplementedError("write your pallas kernel here")


# speedup vs baseline: 4.5711x; 4.5711x over previous
import jax
import jax.numpy as jnp
from jax import lax
from jax.experimental import pallas as pl

K1 = 17
TN = 256


def _edge_kernel(pc_tile_ref, pc_all_ref, ef_ref, idx_ref):
    pc_t = pc_tile_ref[0]
    pc_a = pc_all_ref[0]
    n = pc_a.shape[1]
    tn = pc_t.shape[1]

    sq_t = jnp.sum(pc_t * pc_t, axis=0)
    sq_a = jnp.sum(pc_a * pc_a, axis=0)

    dot = lax.dot_general(pc_t, pc_a, (((0,), (0,)), ((), ())),
                          preferred_element_type=jnp.float32)
    d2 = sq_t[:, None] + sq_a[None, :] - 2.0 * dot

    iota = lax.broadcasted_iota(jnp.int32, (tn, n), 1)

    ef_ref[0, 0:3, :, :] = jnp.broadcast_to(pc_t[:, None, :], (3, K1, tn))

    work = d2
    for k in range(K1):
        m = jnp.min(work, axis=1)
        am = jnp.min(jnp.where(work == m[:, None], iota, n), axis=1)
        sel = iota == am[:, None]
        onehot = sel.astype(jnp.float32)
        nn_k = lax.dot_general(pc_a, onehot, (((1,), (1,)), ((), ())),
                               preferred_element_type=jnp.float32,
                               precision=lax.Precision.HIGHEST)
        work = jnp.where(sel, jnp.float32(jnp.inf), work)
        idx_ref[0, k, :] = am
        ef_ref[0, 3:6, k, :] = nn_k - pc_t


def kernel(point_cloud):
    B, D, N = point_cloud.shape
    grid = (B, N // TN)
    ef, idx = pl.pallas_call(
        _edge_kernel,
        grid=grid,
        in_specs=[
            pl.BlockSpec((1, D, TN), lambda b, j: (b, 0, j)),
            pl.BlockSpec((1, D, N), lambda b, j: (b, 0, 0)),
        ],
        out_specs=[
            pl.BlockSpec((1, 2 * D, K1, TN), lambda b, j: (b, 0, 0, j)),
            pl.BlockSpec((1, K1, TN), lambda b, j: (b, 0, j)),
        ],
        out_shape=[
            jax.ShapeDtypeStruct((B, 2 * D, K1, N), jnp.float32),
            jax.ShapeDtypeStruct((B, K1, N), jnp.int32),
        ],
    )(point_cloud, point_cloud)
    return ef, idx
